# baseline (speedup 1.0000x reference)
import functools

import jax
import jax.numpy as jnp
from jax import lax
from jax.experimental import pallas as pl
from jax.experimental.pallas import tpu as pltpu
from jax.experimental.pallas import tpu_sc as plsc

E = 16
D = 1024
DFF = 4096
N = 4096
T = 256
G = 32
NW = 32
CH = 32


def _routing_body(x_ref, c_ref, eid_ref, alpha_ref):
    aff = lax.dot_general(x_ref[...], c_ref[...],
                          (((1,), (1,)), ((), ())),
                          preferred_element_type=jnp.float32)
    eid_ref[...] = jnp.argmax(aff, axis=1).astype(jnp.int32)
    alpha_ref[...] = jax.nn.sigmoid(jnp.max(aff, axis=1))


def _routing(feats, cents):
    return pl.pallas_call(
        _routing_body,
        out_shape=(jax.ShapeDtypeStruct((N,), jnp.int32),
                   jax.ShapeDtypeStruct((N,), jnp.float32)),
    )(feats, cents)


def _sc_gather(table, idx3, k):
    d = table.shape[1]
    mesh = plsc.VectorSubcoreMesh(core_axis_name="c", subcore_axis_name="s")
    info = plsc.get_sparse_core_info()
    nc = info.num_cores

    @functools.partial(
        pl.kernel, mesh=mesh,
        out_type=jax.ShapeDtypeStruct((NW * k * CH, d), jnp.float32),
        scratch_types=[
            pltpu.VMEM((k, CH), jnp.int32),
            pltpu.VMEM((CH, d), jnp.float32),
            pltpu.VMEM((CH, d), jnp.float32),
            pltpu.SemaphoreType.DMA,
            pltpu.SemaphoreType.DMA,
        ],
    )
    def run(table_hbm, idx_hbm, out_hbm, idx_v, r0, r1, gsem, ssem):
        wid = lax.axis_index("s") * nc + lax.axis_index("c")
        base = wid * k * CH
        pltpu.sync_copy(idx_hbm.at[wid], idx_v)
        bufs = (r0, r1)
        g = [None] * k
        s = [None] * k
        g[0] = pltpu.async_copy(table_hbm.at[idx_v.at[0]], bufs[0], gsem)
        for c in range(k):
            if c + 1 < k:
                if c >= 1:
                    s[c - 1].wait()
                g[c + 1] = pltpu.async_copy(
                    table_hbm.at[idx_v.at[c + 1]], bufs[(c + 1) % 2], gsem)
            g[c].wait()
            s[c] = pltpu.async_copy(
                bufs[c % 2], out_hbm.at[pl.ds(base + c * CH, CH)], ssem)
        s[k - 1].wait()
        if k >= 2:
            s[k - 2].wait()

    return run(table, idx3)


def _ffn_body(off_ref, nt_ref, x_any, a_any, nw_ref, nb_ref,
              w1_ref, b1_ref, w2_ref, b2_ref, y_any,
              xbuf, abuf, ybuf, s_in, s_a, s_out):
    e = pl.program_id(0)
    base = off_ref[e]
    n = nt_ref[e]

    def tile(j, carry):
        row = pl.multiple_of(base + j * T, T)
        cx = pltpu.make_async_copy(x_any.at[pl.ds(row, T)], xbuf, s_in)
        cx.start()
        ca = pltpu.make_async_copy(a_any.at[pl.ds(row, T)], abuf, s_a)
        ca.start()
        cx.wait()
        ca.wait()
        x = xbuf[...]
        mu = jnp.mean(x, axis=1, keepdims=True)
        var = jnp.mean((x - mu) ** 2, axis=1, keepdims=True)
        xh = (x - mu) * lax.rsqrt(var + 1e-5) * nw_ref[0] + nb_ref[0]
        h = lax.dot_general(xh.astype(jnp.bfloat16), w1_ref[0],
                            (((1,), (1,)), ((), ())),
                            preferred_element_type=jnp.float32)
        h = jnp.maximum(h + b1_ref[0], 0.0)
        y = lax.dot_general(h.astype(jnp.bfloat16), w2_ref[0],
                            (((1,), (1,)), ((), ())),
                            preferred_element_type=jnp.float32)
        ybuf[...] = x + abuf[...] * (y + b2_ref[0])
        co = pltpu.make_async_copy(ybuf, y_any.at[pl.ds(row, T)], s_out)
        co.start()
        co.wait()
        return carry

    lax.fori_loop(0, n, tile, 0)


def _grouped_ffn(off, nt, x_p, alpha_p, norm_w, norm_b, ff1_w, ff1_b, ff2_w, ff2_b):
    hbm = pl.BlockSpec(memory_space=pltpu.MemorySpace.HBM)
    grid_spec = pltpu.PrefetchScalarGridSpec(
        num_scalar_prefetch=2,
        grid=(E,),
        in_specs=[
            hbm,
            hbm,
            pl.BlockSpec((1, 1, D), lambda e, off, nt: (e, 0, 0)),
            pl.BlockSpec((1, 1, D), lambda e, off, nt: (e, 0, 0)),
            pl.BlockSpec((1, DFF, D), lambda e, off, nt: (e, 0, 0)),
            pl.BlockSpec((1, 1, DFF), lambda e, off, nt: (e, 0, 0)),
            pl.BlockSpec((1, D, DFF), lambda e, off, nt: (e, 0, 0)),
            pl.BlockSpec((1, 1, D), lambda e, off, nt: (e, 0, 0)),
        ],
        out_specs=hbm,
        scratch_shapes=[
            pltpu.VMEM((T, D), jnp.float32),
            pltpu.VMEM((T, 1), jnp.float32),
            pltpu.VMEM((T, D), jnp.float32),
            pltpu.SemaphoreType.DMA,
            pltpu.SemaphoreType.DMA,
            pltpu.SemaphoreType.DMA,
        ],
    )
    return pl.pallas_call(
        _ffn_body,
        grid_spec=grid_spec,
        out_shape=jax.ShapeDtypeStruct((G * T, D), jnp.float32),
        compiler_params=pltpu.CompilerParams(
            dimension_semantics=("arbitrary",)),
    )(off, nt, x_p, alpha_p, norm_w, norm_b, ff1_w, ff1_b, ff2_w, ff2_b)


def kernel(input_features, expert_centroids, norm_w, norm_b,
           ff1_w, ff1_b, ff2_w, ff2_b):
    feats = input_features.reshape(N, D)

    eid, alpha = _routing(feats, expert_centroids)

    oh = (eid[:, None] == jnp.arange(E, dtype=jnp.int32)[None, :]).astype(jnp.int32)
    csum = jnp.cumsum(oh, axis=0)
    rank = jnp.sum((csum - oh) * oh, axis=1)
    counts = csum[-1]
    tiles_e = (counts + T - 1) // T
    tile_start = jnp.cumsum(tiles_e) - tiles_e
    pos = tile_start[eid] * T + rank
    src_idx = (jnp.arange(G * T, dtype=jnp.int32) % N).at[pos].set(
        jnp.arange(N, dtype=jnp.int32))
    alpha_p = jnp.zeros(G * T, jnp.float32).at[pos].set(alpha)
    off = (tile_start * T).astype(jnp.int32)
    nt = tiles_e.astype(jnp.int32)

    x_p = _sc_gather(feats, src_idx.reshape(NW, (G * T) // (NW * CH), CH),
                     (G * T) // (NW * CH))
    y_p = _grouped_ffn(off, nt, x_p, alpha_p.reshape(G * T, 1),
                       norm_w.reshape(E, 1, D), norm_b.reshape(E, 1, D),
                       ff1_w.astype(jnp.bfloat16), ff1_b.reshape(E, 1, DFF),
                       ff2_w.astype(jnp.bfloat16), ff2_b.reshape(E, 1, D))
    out = _sc_gather(y_p, pos.reshape(NW, N // (NW * CH), CH),
                     N // (NW * CH))
    return out.reshape(input_features.shape)


# speedup vs baseline: 1.1678x; 1.1678x over previous
import functools

import jax
import jax.numpy as jnp
from jax import lax
from jax.experimental import pallas as pl
from jax.experimental.pallas import tpu as pltpu
from jax.experimental.pallas import tpu_sc as plsc

E = 16
D = 1024
DFF = 4096
N = 4096
T = 256
G = 32
NW = 32
CH = 32


def _routing_body(x_ref, c_ref, eid_ref, alpha_ref):
    aff = lax.dot_general(x_ref[...], c_ref[...],
                          (((1,), (1,)), ((), ())),
                          preferred_element_type=jnp.float32)
    eid_ref[...] = jnp.argmax(aff, axis=1).astype(jnp.int32)
    alpha_ref[...] = jax.nn.sigmoid(jnp.max(aff, axis=1))


def _routing(feats, cents):
    return pl.pallas_call(
        _routing_body,
        out_shape=(jax.ShapeDtypeStruct((N,), jnp.int32),
                   jax.ShapeDtypeStruct((N,), jnp.float32)),
    )(feats, cents)


def _sc_gather(table, idx3, k):
    d = table.shape[1]
    mesh = plsc.VectorSubcoreMesh(core_axis_name="c", subcore_axis_name="s")
    info = plsc.get_sparse_core_info()
    nc = info.num_cores

    @functools.partial(
        pl.kernel, mesh=mesh,
        out_type=jax.ShapeDtypeStruct((NW * k * CH, d), jnp.float32),
        scratch_types=[
            pltpu.VMEM((k, CH), jnp.int32),
            pltpu.VMEM((CH, d), jnp.float32),
            pltpu.VMEM((CH, d), jnp.float32),
            pltpu.SemaphoreType.DMA,
            pltpu.SemaphoreType.DMA,
        ],
    )
    def run(table_hbm, idx_hbm, out_hbm, idx_v, r0, r1, gsem, ssem):
        wid = lax.axis_index("s") * nc + lax.axis_index("c")
        base = wid * k * CH
        pltpu.sync_copy(idx_hbm.at[wid], idx_v)
        bufs = (r0, r1)
        g = [None] * k
        s = [None] * k
        g[0] = pltpu.async_copy(table_hbm.at[idx_v.at[0]], bufs[0], gsem)
        for c in range(k):
            if c + 1 < k:
                if c >= 1:
                    s[c - 1].wait()
                g[c + 1] = pltpu.async_copy(
                    table_hbm.at[idx_v.at[c + 1]], bufs[(c + 1) % 2], gsem)
            g[c].wait()
            s[c] = pltpu.async_copy(
                bufs[c % 2], out_hbm.at[pl.ds(base + c * CH, CH)], ssem)
        s[k - 1].wait()
        if k >= 2:
            s[k - 2].wait()

    return run(table, idx3)


def _ffn_body(te_ref, act_ref, chg_ref, fnx_ref, sl_ref,
              x_ref, a_ref, nw_ref, nb_ref, b1_ref, b2_ref,
              w1_hbm, w2_hbm, out_ref, w1buf, w2buf, wsem):
    g = pl.program_id(0)
    e = te_ref[g]
    sl = sl_ref[g]

    @pl.when(g == 0)
    def _():
        pltpu.make_async_copy(w1_hbm.at[e], w1buf.at[0], wsem).start()
        pltpu.make_async_copy(w2_hbm.at[e], w2buf.at[0], wsem).start()

    @pl.when(chg_ref[g] == 1)
    def _():
        pltpu.make_async_copy(w1_hbm.at[e], w1buf.at[sl], wsem).wait()
        pltpu.make_async_copy(w2_hbm.at[e], w2buf.at[sl], wsem).wait()

    @pl.when(fnx_ref[g] == 1)
    def _():
        ne = te_ref[g + 1]
        pltpu.make_async_copy(w1_hbm.at[ne], w1buf.at[1 - sl], wsem).start()
        pltpu.make_async_copy(w2_hbm.at[ne], w2buf.at[1 - sl], wsem).start()

    @pl.when(g < act_ref[0])
    def _():
        x = x_ref[...]
        mu = jnp.mean(x, axis=1, keepdims=True)
        var = jnp.mean((x - mu) ** 2, axis=1, keepdims=True)
        xh = (x - mu) * lax.rsqrt(var + 1e-5) * nw_ref[0] + nb_ref[0]
        h = lax.dot_general(xh.astype(jnp.bfloat16), w1buf[sl],
                            (((1,), (1,)), ((), ())),
                            preferred_element_type=jnp.float32)
        h = jnp.maximum(h + b1_ref[0], 0.0)
        y = lax.dot_general(h.astype(jnp.bfloat16), w2buf[sl],
                            (((1,), (1,)), ((), ())),
                            preferred_element_type=jnp.float32)
        out_ref[...] = x + a_ref[...] * (y + b2_ref[0])


def _grouped_ffn(te, act, chg, fnx, wslot, x_p, alpha_p,
                 norm_w, norm_b, ff1_w, ff1_b, ff2_w, ff2_b):
    hbm = pl.BlockSpec(memory_space=pltpu.MemorySpace.HBM)
    grid_spec = pltpu.PrefetchScalarGridSpec(
        num_scalar_prefetch=5,
        grid=(G,),
        in_specs=[
            pl.BlockSpec((T, D), lambda g, *s: (g, 0)),
            pl.BlockSpec((T, 1), lambda g, *s: (g, 0)),
            pl.BlockSpec((1, 1, D), lambda g, te, *s: (te[g], 0, 0)),
            pl.BlockSpec((1, 1, D), lambda g, te, *s: (te[g], 0, 0)),
            pl.BlockSpec((1, 1, DFF), lambda g, te, *s: (te[g], 0, 0)),
            pl.BlockSpec((1, 1, D), lambda g, te, *s: (te[g], 0, 0)),
            hbm,
            hbm,
        ],
        out_specs=pl.BlockSpec((T, D), lambda g, *s: (g, 0)),
        scratch_shapes=[
            pltpu.VMEM((2, DFF, D), jnp.bfloat16),
            pltpu.VMEM((2, D, DFF), jnp.bfloat16),
            pltpu.SemaphoreType.DMA,
        ],
    )
    return pl.pallas_call(
        _ffn_body,
        grid_spec=grid_spec,
        out_shape=jax.ShapeDtypeStruct((G * T, D), jnp.float32),
        compiler_params=pltpu.CompilerParams(
            dimension_semantics=("arbitrary",)),
    )(te, act, chg, fnx, wslot, x_p, alpha_p,
      norm_w, norm_b, ff1_b, ff2_b, ff1_w, ff2_w)


def kernel(input_features, expert_centroids, norm_w, norm_b,
           ff1_w, ff1_b, ff2_w, ff2_b):
    feats = input_features.reshape(N, D)

    eid, alpha = _routing(feats, expert_centroids)

    oh = (eid[:, None] == jnp.arange(E, dtype=jnp.int32)[None, :]).astype(jnp.int32)
    csum = jnp.cumsum(oh, axis=0)
    rank = jnp.sum((csum - oh) * oh, axis=1)
    counts = csum[-1]
    tiles_e = (counts + T - 1) // T
    tile_start = jnp.cumsum(tiles_e) - tiles_e
    pos = tile_start[eid] * T + rank
    src_idx = (jnp.arange(G * T, dtype=jnp.int32) % N).at[pos].set(
        jnp.arange(N, dtype=jnp.int32))
    alpha_p = jnp.zeros(G * T, jnp.float32).at[pos].set(alpha)
    tcum = jnp.cumsum(tiles_e)
    n_act = tcum[-1]
    gidx = jnp.arange(G, dtype=jnp.int32)
    te_raw = jnp.minimum(
        jnp.searchsorted(tcum, gidx, side="right"), E - 1).astype(jnp.int32)
    last_e = te_raw[jnp.maximum(n_act - 1, 0)]
    te = jnp.where(gidx < n_act, te_raw, last_e)
    act = n_act.reshape(1).astype(jnp.int32)
    prev = jnp.concatenate([jnp.full((1,), -1, jnp.int32), te[:-1]])
    chg = (te != prev).astype(jnp.int32)
    nxt = jnp.concatenate([te[1:], te[-1:]])
    fnx = ((nxt != te) & (gidx + 1 < G)).astype(jnp.int32)
    wslot = ((jnp.cumsum(chg) - 1) % 2).astype(jnp.int32)

    x_p = _sc_gather(feats, src_idx.reshape(NW, (G * T) // (NW * CH), CH),
                     (G * T) // (NW * CH))
    y_p = _grouped_ffn(te, act, chg, fnx, wslot, x_p, alpha_p.reshape(G * T, 1),
                       norm_w.reshape(E, 1, D), norm_b.reshape(E, 1, D),
                       ff1_w.astype(jnp.bfloat16), ff1_b.reshape(E, 1, DFF),
                       ff2_w.astype(jnp.bfloat16), ff2_b.reshape(E, 1, D))
    out = _sc_gather(y_p, pos.reshape(NW, N // (NW * CH), CH),
                     N // (NW * CH))
    return out.reshape(input_features.shape)


# speedup vs baseline: 1.2687x; 1.0864x over previous
import functools

import jax
import jax.numpy as jnp
from jax import lax
from jax.experimental import pallas as pl
from jax.experimental.pallas import tpu as pltpu
from jax.experimental.pallas import tpu_sc as plsc

E = 16
D = 1024
DFF = 4096
N = 4096
T = 256
G = 32
NW = 32
CH = 32
AW = 128


TR = 512
NR = N // TR


def _routing_body(x_ref, c_ref, eid_ref, alpha_ref, rank_ref, counts_ref,
                  carry):
    g = pl.program_id(0)

    @pl.when(g == 0)
    def _():
        carry[...] = jnp.zeros_like(carry)

    aff = lax.dot_general(x_ref[...], c_ref[...],
                          (((1,), (1,)), ((), ())),
                          preferred_element_type=jnp.float32)
    eid = jnp.argmax(aff, axis=1).astype(jnp.int32)
    eid_ref[...] = eid
    alpha_ref[...] = jax.nn.sigmoid(jnp.max(aff, axis=1))
    ohf = (eid[:, None] == lax.broadcasted_iota(jnp.int32, (1, E), 1)
           ).astype(jnp.float32)
    ii = lax.broadcasted_iota(jnp.int32, (TR, TR), 0)
    jj = lax.broadcasted_iota(jnp.int32, (TR, TR), 1)
    tri = (ii > jj).astype(jnp.float32)
    within = lax.dot_general(tri, ohf, (((1,), (0,)), ((), ())),
                             preferred_element_type=jnp.float32)
    base = carry[...]
    rank_ref[...] = jnp.sum((within + base) * ohf, axis=1).astype(jnp.int32)
    carry[...] = base + jnp.sum(ohf, axis=0, keepdims=True)

    @pl.when(g == NR - 1)
    def _():
        counts_ref[...] = carry[...].astype(jnp.int32)


def _routing(feats, cents):
    return pl.pallas_call(
        _routing_body,
        grid=(NR,),
        in_specs=[
            pl.BlockSpec((TR, D), lambda g: (g, 0)),
            pl.BlockSpec((E, D), lambda g: (0, 0)),
        ],
        out_specs=(
            pl.BlockSpec((TR,), lambda g: (g,)),
            pl.BlockSpec((TR,), lambda g: (g,)),
            pl.BlockSpec((TR,), lambda g: (g,)),
            pl.BlockSpec((1, E), lambda g: (0, 0)),
        ),
        out_shape=(jax.ShapeDtypeStruct((N,), jnp.int32),
                   jax.ShapeDtypeStruct((N,), jnp.float32),
                   jax.ShapeDtypeStruct((N,), jnp.int32),
                   jax.ShapeDtypeStruct((1, E), jnp.int32)),
        scratch_shapes=[pltpu.VMEM((1, E), jnp.float32)],
        compiler_params=pltpu.CompilerParams(
            dimension_semantics=("arbitrary",)),
    )(feats, cents)


def _sc_scatter(feats, alpha16, pos3, k):
    mesh = plsc.VectorSubcoreMesh(core_axis_name="c", subcore_axis_name="s")
    info = plsc.get_sparse_core_info()
    nc = info.num_cores

    @functools.partial(
        pl.kernel, mesh=mesh,
        out_type=(jax.ShapeDtypeStruct((G * T, D), jnp.float32),
                  jax.ShapeDtypeStruct((G * T, AW), jnp.float32)),
        scratch_types=[
            pltpu.VMEM((k, CH), jnp.int32),
            pltpu.VMEM((CH, D), jnp.float32),
            pltpu.VMEM((CH, AW), jnp.float32),
            pltpu.SemaphoreType.DMA,
            pltpu.SemaphoreType.DMA,
        ],
    )
    def run(f_hbm, a_hbm, pos_hbm, xp_hbm, ap_hbm, posv, xb, ab, s1, s2):
        wid = lax.axis_index("s") * nc + lax.axis_index("c")
        base = wid * k * CH
        pltpu.sync_copy(pos_hbm.at[wid], posv)
        for c in range(k):
            pltpu.sync_copy(f_hbm.at[pl.ds(base + c * CH, CH)], xb)
            pltpu.sync_copy(a_hbm.at[pl.ds(base + c * CH, CH)], ab)
            cx = pltpu.async_copy(xb, xp_hbm.at[posv.at[c]], s1)
            ca = pltpu.async_copy(ab, ap_hbm.at[posv.at[c]], s2)
            cx.wait()
            ca.wait()

    return run(feats, alpha16, pos3)


def _sc_gather(table, idx3, k):
    d = table.shape[1]
    mesh = plsc.VectorSubcoreMesh(core_axis_name="c", subcore_axis_name="s")
    info = plsc.get_sparse_core_info()
    nc = info.num_cores

    @functools.partial(
        pl.kernel, mesh=mesh,
        out_type=jax.ShapeDtypeStruct((NW * k * CH, d), jnp.float32),
        scratch_types=[
            pltpu.VMEM((k, CH), jnp.int32),
            pltpu.VMEM((CH, d), jnp.float32),
            pltpu.VMEM((CH, d), jnp.float32),
            pltpu.SemaphoreType.DMA,
            pltpu.SemaphoreType.DMA,
        ],
    )
    def run(table_hbm, idx_hbm, out_hbm, idx_v, r0, r1, gsem, ssem):
        wid = lax.axis_index("s") * nc + lax.axis_index("c")
        base = wid * k * CH
        pltpu.sync_copy(idx_hbm.at[wid], idx_v)
        bufs = (r0, r1)
        g = [None] * k
        s = [None] * k
        g[0] = pltpu.async_copy(table_hbm.at[idx_v.at[0]], bufs[0], gsem)
        for c in range(k):
            if c + 1 < k:
                if c >= 1:
                    s[c - 1].wait()
                g[c + 1] = pltpu.async_copy(
                    table_hbm.at[idx_v.at[c + 1]], bufs[(c + 1) % 2], gsem)
            g[c].wait()
            s[c] = pltpu.async_copy(
                bufs[c % 2], out_hbm.at[pl.ds(base + c * CH, CH)], ssem)
        s[k - 1].wait()
        if k >= 2:
            s[k - 2].wait()

    return run(table, idx3)


def _ffn_body(te_ref, act_ref, chg_ref, fnx_ref, sl_ref,
              x_ref, a_ref, nw_ref, nb_ref, b1_ref, b2_ref,
              w1_hbm, w2_hbm, out_ref, w1buf, w2buf, wsem):
    g = pl.program_id(0)
    e = te_ref[g]
    sl = sl_ref[g]

    @pl.when(g == 0)
    def _():
        pltpu.make_async_copy(w1_hbm.at[e], w1buf.at[0], wsem).start()
        pltpu.make_async_copy(w2_hbm.at[e], w2buf.at[0], wsem).start()

    @pl.when(chg_ref[g] == 1)
    def _():
        pltpu.make_async_copy(w1_hbm.at[e], w1buf.at[sl], wsem).wait()
        pltpu.make_async_copy(w2_hbm.at[e], w2buf.at[sl], wsem).wait()

    @pl.when(fnx_ref[g] == 1)
    def _():
        ne = te_ref[g + 1]
        pltpu.make_async_copy(w1_hbm.at[ne], w1buf.at[1 - sl], wsem).start()
        pltpu.make_async_copy(w2_hbm.at[ne], w2buf.at[1 - sl], wsem).start()

    @pl.when(g < act_ref[0])
    def _():
        x = x_ref[...]
        a = a_ref[...][:, 0:1]
        mu = jnp.mean(x, axis=1, keepdims=True)
        var = jnp.mean((x - mu) ** 2, axis=1, keepdims=True)
        xh = (x - mu) * lax.rsqrt(var + 1e-5) * nw_ref[0] + nb_ref[0]
        h = lax.dot_general(xh.astype(jnp.bfloat16), w1buf[sl],
                            (((1,), (1,)), ((), ())),
                            preferred_element_type=jnp.float32)
        h = jnp.maximum(h + b1_ref[0], 0.0)
        y = lax.dot_general(h.astype(jnp.bfloat16), w2buf[sl],
                            (((1,), (1,)), ((), ())),
                            preferred_element_type=jnp.float32)
        out_ref[...] = x + a * (y + b2_ref[0])


def _grouped_ffn(te, act, chg, fnx, wslot, x_p, alpha_p,
                 norm_w, norm_b, ff1_w, ff1_b, ff2_w, ff2_b):
    hbm = pl.BlockSpec(memory_space=pltpu.MemorySpace.HBM)
    grid_spec = pltpu.PrefetchScalarGridSpec(
        num_scalar_prefetch=5,
        grid=(G,),
        in_specs=[
            pl.BlockSpec((T, D), lambda g, *s: (g, 0)),
            pl.BlockSpec((T, AW), lambda g, *s: (g, 0)),
            pl.BlockSpec((1, 1, D), lambda g, te, *s: (te[g], 0, 0)),
            pl.BlockSpec((1, 1, D), lambda g, te, *s: (te[g], 0, 0)),
            pl.BlockSpec((1, 1, DFF), lambda g, te, *s: (te[g], 0, 0)),
            pl.BlockSpec((1, 1, D), lambda g, te, *s: (te[g], 0, 0)),
            hbm,
            hbm,
        ],
        out_specs=pl.BlockSpec((T, D), lambda g, *s: (g, 0)),
        scratch_shapes=[
            pltpu.VMEM((2, DFF, D), jnp.bfloat16),
            pltpu.VMEM((2, D, DFF), jnp.bfloat16),
            pltpu.SemaphoreType.DMA,
        ],
    )
    return pl.pallas_call(
        _ffn_body,
        grid_spec=grid_spec,
        out_shape=jax.ShapeDtypeStruct((G * T, D), jnp.float32),
        compiler_params=pltpu.CompilerParams(
            dimension_semantics=("arbitrary",)),
    )(te, act, chg, fnx, wslot, x_p, alpha_p,
      norm_w, norm_b, ff1_b, ff2_b, ff1_w, ff2_w)


def kernel(input_features, expert_centroids, norm_w, norm_b,
           ff1_w, ff1_b, ff2_w, ff2_b):
    feats = input_features.reshape(N, D)

    eid, alpha, rank, counts2 = _routing(feats, expert_centroids)

    counts = counts2[0]
    tiles_e = (counts + T - 1) // T
    tcum = jnp.cumsum(tiles_e)
    tile_start = tcum - tiles_e
    pad_off = (tile_start * T).astype(jnp.int32)
    ar16 = jnp.arange(E, dtype=jnp.int32)
    pos = rank + jnp.sum(
        (eid[:, None] == ar16[None, :]) * pad_off[None, :], axis=1,
        dtype=jnp.int32)
    alpha16 = jnp.broadcast_to(alpha[:, None], (N, AW))
    n_act = tcum[-1]
    gidx = jnp.arange(G, dtype=jnp.int32)
    te_raw = jnp.minimum(
        jnp.searchsorted(tcum, gidx, side="right"), E - 1).astype(jnp.int32)
    last_e = te_raw[jnp.maximum(n_act - 1, 0)]
    te = jnp.where(gidx < n_act, te_raw, last_e)
    act = n_act.reshape(1).astype(jnp.int32)
    prev = jnp.concatenate([jnp.full((1,), -1, jnp.int32), te[:-1]])
    chg = (te != prev).astype(jnp.int32)
    nxt = jnp.concatenate([te[1:], te[-1:]])
    fnx = ((nxt != te) & (gidx + 1 < G)).astype(jnp.int32)
    wslot = ((jnp.cumsum(chg) - 1) % 2).astype(jnp.int32)

    x_p, a_p = _sc_scatter(feats, alpha16,
                           pos.reshape(NW, N // (NW * CH), CH),
                           N // (NW * CH))
    y_p = _grouped_ffn(te, act, chg, fnx, wslot, x_p, a_p,
                       norm_w.reshape(E, 1, D), norm_b.reshape(E, 1, D),
                       ff1_w.astype(jnp.bfloat16), ff1_b.reshape(E, 1, DFF),
                       ff2_w.astype(jnp.bfloat16), ff2_b.reshape(E, 1, D))
    out = _sc_gather(y_p, pos.reshape(NW, N // (NW * CH), CH),
                     N // (NW * CH))
    return out.reshape(input_features.shape)
